# baseline (speedup 1.0000x reference)
import numpy as np
import jax
import jax.numpy as jnp
from jax import lax
from jax.experimental import pallas as pl
from jax.experimental.pallas import tpu as pltpu

_BLADES = [(), (0,), (1,), (2,), (3,),
           (0, 1), (0, 2), (0, 3), (1, 2), (1, 3), (2, 3),
           (0, 1, 2), (0, 1, 3), (0, 2, 3), (1, 2, 3),
           (0, 1, 2, 3)]
_IDX = {b: i for i, b in enumerate(_BLADES)}


def _blade_mul(a, b):
    lst = list(a) + list(b); sign = 1; n = len(lst)
    for _ in range(n):
        for j in range(n - 1):
            if lst[j] > lst[j + 1]:
                lst[j], lst[j + 1] = lst[j + 1], lst[j]; sign = -sign
    out, i = [], 0
    while i < len(lst):
        if i + 1 < len(lst) and lst[i] == lst[i + 1]:
            if lst[i] == 0:
                return 0, ()
            i += 2
        else:
            out.append(lst[i]); i += 1
    return sign, tuple(out)


_GPn = np.zeros((16, 16, 16), dtype=np.float32)
for _i, _a in enumerate(_BLADES):
    for _j, _b in enumerate(_BLADES):
        _s, _c = _blade_mul(_a, _b)
        if _s:
            _GPn[_i, _j, _IDX[_c]] = _s
_gr = np.array([len(b) for b in _BLADES])
_P = np.stack([np.diag((_gr == g).astype(np.float32)) for g in range(5)])
_L0 = _GPn[1].T
_BASISn = np.concatenate([_P, np.stack([_L0 @ _P[g] for g in range(4)])], 0)
_BASIS = jnp.asarray(_BASISn)
_INNERn = np.array([0.0 if 0 in b else 1.0 for b in _BLADES], np.float32)
_NONE0 = [i for i, b in enumerate(_BLADES) if 0 not in b]

_GP_TERMS = [[] for _ in range(16)]
for _i in range(16):
    for _j in range(16):
        _nz = np.nonzero(_GPn[_i, _j])[0]
        if _nz.size:
            _GP_TERMS[int(_nz[0])].append((_i, _j, float(_GPn[_i, _j, _nz[0]])))

_C, _S, _H = 16, 32, 8
_HQ, _HV = 24, 40
_SCALE = 1.0 / np.sqrt(8 * (_C // _H) + _S // _H)
_PREC = lax.Precision.HIGHEST

_r = np.arange(256)
_PERM_AO = (_r % 16) * 16 + _r // 16
_IDX_OUT288 = np.concatenate([_PERM_AO, 256 + np.arange(32)]).astype(np.int32)

_r = np.arange(512)
_IDX_M1 = np.concatenate([(_r % 32) * 16 + _r // 32,
                          512 + np.arange(64)]).astype(np.int32)

_IDX_QKV = np.full((704,), 768 + 96, np.int32)
for _h in range(8):
    for _ci in range(2):
        for _t in range(8):
            _IDX_QKV[_h * 24 + _ci * 8 + _t] = (2 * _h + _ci) * 16 + _NONE0[_t]
            _IDX_QKV[192 + _h * 24 + _ci * 8 + _t] = (16 + 2 * _h + _ci) * 16 + _NONE0[_t]
        for _b in range(16):
            _IDX_QKV[384 + _h * 40 + _ci * 16 + _b] = (32 + 2 * _h + _ci) * 16 + _b
    for _d in range(4):
        _IDX_QKV[_h * 24 + 16 + _d] = 768 + _h * 4 + _d
        _IDX_QKV[192 + _h * 24 + 16 + _d] = 768 + 32 + _h * 4 + _d
        _IDX_QKV[384 + _h * 40 + 32 + _d] = 768 + 64 + _h * 4 + _d

_COL_AO = np.zeros((288,), np.int32)
for _h in range(8):
    for _ci in range(2):
        for _b in range(16):
            _COL_AO[_h * 36 + _ci * 16 + _b] = _b * 16 + (2 * _h + _ci)
    for _d in range(4):
        _COL_AO[_h * 36 + 32 + _d] = 256 + _h * 4 + _d

_QSCALEn = np.ones((1, 704, 1), np.float32)
_QSCALEn[:, :192, :] = _SCALE

_MASKROWn = np.repeat(_INNERn, 16).astype(np.float32)
_MASK2Dn = np.broadcast_to(_MASKROWn[:, None], (256, 128)).copy()


def _eq_canon(wmv, wsm, wms, wss):
    L, O, I, _ = wmv.shape
    So, Si = wss.shape[1], wss.shape[2]
    Wf = jnp.einsum('loik,kab->loabi', wmv, _BASIS,
                    precision=lax.Precision.HIGHEST)
    cx_mv = Wf.reshape(L, O * 16, 16 * I)
    cx_s = jnp.zeros((L, So, 16, I), wms.dtype).at[:, :, 0, :].set(wms)
    cx = jnp.concatenate([cx_mv, cx_s.reshape(L, So, 16 * I)], 1)
    cs_mv = jnp.zeros((L, O, 16, Si), wsm.dtype).at[:, :, 0, :].set(wsm)
    cs = jnp.concatenate([cs_mv.reshape(L, O * 16, Si), wss], 1)
    return cx, cs


def _take_rows(m, idx):
    z = jnp.concatenate([m, jnp.zeros_like(m[:, :1])], 1)
    return jnp.take(z, jnp.asarray(idx), axis=1)


def _take_cols(m, idx):
    z = jnp.concatenate([m, jnp.zeros_like(m[:, :, :1])], 2)
    return jnp.take(z, jnp.asarray(idx), axis=2)


def _body(inp_ref, bs_ref, a8_ref, mask_ref,
          wqkvxh_ref, wqkvxl_ref, wqkvs_ref, waoh_ref, waol_ref,
          wm1xh_ref, wm1xl_ref, wm1s_ref, wm2xh_ref, wm2xl_ref,
          wm2s_ref, rx_ref, rs_ref, out_ref):
    NPT = inp_ref.shape[2]
    L = wqkvs_ref.shape[0]
    f32 = jnp.float32
    bf16 = jnp.bfloat16

    def mm(a, b):
        return lax.dot_general(a, b, (((1,), (0,)), ((), ())),
                               precision=_PREC, preferred_element_type=f32)

    def mmT(a, b):
        return lax.dot_general(a, b, (((0,), (0,)), ((), ())),
                               precision=_PREC, preferred_element_type=f32)

    def split(x):
        hi = x.astype(bf16)
        lo = (x - hi.astype(f32)).astype(bf16)
        return hi, lo

    def bdot(a, b):
        return lax.dot_general(a, b, (((1,), (0,)), ((), ())),
                               preferred_element_type=f32)

    def bdotT(a, b):
        return lax.dot_general(a, b, (((0,), (0,)), ((), ())),
                               preferred_element_type=f32)

    def mm3(ah, al, b):
        bh, bl = split(b)
        return (bdot(ah, bl) + bdot(al, bh)) + bdot(ah, bh)

    def mm3T(a, b):
        ah, al = split(a)
        bh, bl = split(b)
        return (bdotT(ah, bl) + bdotT(al, bh)) + bdotT(ah, bh)

    mask1 = mask_ref[:, 0:1]

    def ln(X, Sc):
        f = jnp.sum(X * X * mask1, axis=0, keepdims=True) * (1.0 / 16.0)
        g = jnp.sum(Sc * Sc, axis=0, keepdims=True) * (1.0 / 32.0)
        return X * lax.rsqrt(f + 1e-6), Sc * lax.rsqrt(g + 1e-6)

    X = mm(a8_ref[...], inp_ref[0])
    Sc = bs_ref[...]

    def layer(l, carry):
        X, Sc = carry
        Xn, Sn = ln(X, Sc)
        qkv = mm3(wqkvxh_ref[l], wqkvxl_ref[l], Xn) \
            + mm(wqkvs_ref[l], Sn)
        outs = []
        for h in range(_H):
            q = lax.slice(qkv, (h * _HQ, 0), (h * _HQ + 20, NPT))
            k = lax.slice(qkv, (192 + h * _HQ, 0), (192 + h * _HQ + 20, NPT))
            v = lax.slice(qkv, (384 + h * _HV, 0), (384 + h * _HV + 36, NPT))
            qn2 = jnp.sum(q * q, axis=0, keepdims=True)
            kn2 = jnp.sum(k * k, axis=0, keepdims=True)
            bnd = jnp.sqrt(qn2 * jnp.max(kn2))
            q1 = jnp.concatenate([q, -bnd], axis=0)
            k1 = jnp.concatenate([k, jnp.ones((1, NPT), f32)], axis=0)
            p = jnp.exp(mm3T(k1, q1))
            ph, pl_ = split(p)
            vh, vl = split(v)
            one = jnp.ones((1, NPT), bf16)
            zero = jnp.zeros((1, NPT), bf16)
            vh = jnp.concatenate([vh, one], axis=0)
            vl = jnp.concatenate([vl, zero], axis=0)
            o = (bdot(vh, pl_) + bdot(vl, ph)) + bdot(vh, ph)
            den = lax.slice(o, (36, 0), (37, NPT))
            o = lax.slice(o, (0, 0), (36, NPT)) * (1.0 / den)
            outs.append(o)
        O = jnp.concatenate(outs, axis=0)
        D = mm3(waoh_ref[l], waol_ref[l], O)
        X = X + lax.slice(D, (0, 0), (256, NPT))
        Sc = Sc + lax.slice(D, (256, 0), (288, NPT))

        Xn, Sn = ln(X, Sc)
        H1 = mm3(wm1xh_ref[l], wm1xl_ref[l], Xn) \
            + mm(wm1s_ref[l], Sn)
        gps = []
        for kk in range(16):
            acc = None
            for (i, j, s) in _GP_TERMS[kk]:
                t = (lax.slice(H1, (i * 32, 0), (i * 32 + 16, NPT))
                     * lax.slice(H1, (j * 32 + 16, 0), (j * 32 + 32, NPT)))
                t = t if s > 0 else -t
                acc = t if acc is None else acc + t
            gps.append(acc)
        gate = jax.nn.gelu(gps[0])
        gp = jnp.concatenate([g * gate for g in gps], axis=0)
        sh = (lax.slice(H1, (512, 0), (544, NPT))
              * jax.nn.gelu(lax.slice(H1, (544, 0), (576, NPT))))
        D = mm3(wm2xh_ref[l], wm2xl_ref[l], gp) \
            + mm(wm2s_ref[l], sh)
        X = X + lax.slice(D, (0, 0), (256, NPT))
        Sc = Sc + lax.slice(D, (256, 0), (288, NPT))
        return (X, Sc)

    X, Sc = lax.fori_loop(0, L, layer, (X, Sc))
    val = mm(rx_ref[...], X) + mm(rs_ref[...], Sc)
    mean = jnp.sum(lax.slice(val, (0, 0), (1, NPT)), axis=1,
                   keepdims=True) * (1.0 / NPT)
    out_ref[0] = jnp.broadcast_to(mean, (1, 128))


def kernel(inputs, win_mv, win_ms, win_bs,
           a_qkv_wmv, a_qkv_wsm, a_qkv_wms, a_qkv_wss,
           a_out_wmv, a_out_wsm, a_out_wms, a_out_wss,
           m1_wmv, m1_wsm, m1_wms, m1_wss,
           m2_wmv, m2_wsm, m2_wms, m2_wss,
           wout_mv, wout_sm):
    f32 = jnp.float32
    B, NPT, _ = inputs.shape
    L = a_qkv_wmv.shape[0]

    def wsplit(w):
        hi = w.astype(jnp.bfloat16)
        lo = (w - hi.astype(f32)).astype(jnp.bfloat16)
        return hi, lo

    cx, cs = _eq_canon(a_qkv_wmv, a_qkv_wsm, a_qkv_wms, a_qkv_wss)
    wqkv_xh, wqkv_xl = wsplit(_take_rows(cx, _IDX_QKV) * jnp.asarray(_QSCALEn))
    wqkv_s = _take_rows(cs, _IDX_QKV) * jnp.asarray(_QSCALEn)

    cx, cs = _eq_canon(a_out_wmv, a_out_wsm, a_out_wms, a_out_wss)
    w_full = jnp.concatenate([cx, cs], axis=2)
    wao_h, wao_l = wsplit(
        _take_cols(_take_rows(w_full, _IDX_OUT288), _COL_AO))

    cx, cs = _eq_canon(m1_wmv, m1_wsm, m1_wms, m1_wss)
    wm1_xh, wm1_xl = wsplit(_take_rows(cx, _IDX_M1))
    wm1_s = _take_rows(cs, _IDX_M1)

    cx, cs = _eq_canon(m2_wmv, m2_wsm, m2_wms, m2_wss)
    wm2_xh, wm2_xl = wsplit(_take_rows(cx, _IDX_OUT288))
    wm2_s = _take_rows(cs, _IDX_OUT288)

    W2d = jnp.einsum('ok,kab->aob', win_mv[:, 0, :], _BASIS,
                     precision=lax.Precision.HIGHEST).reshape(256, 16)
    A8 = jnp.stack([-W2d[:, 13], W2d[:, 12], -W2d[:, 11], W2d[:, 14]]
                   + [jnp.zeros((256,), f32)] * 4, axis=1)
    inpP = jnp.concatenate([jnp.swapaxes(inputs, 1, 2),
                            jnp.ones((B, 1, NPT), f32),
                            jnp.zeros((B, 4, NPT), f32)], axis=1)
    bs2d = jnp.broadcast_to(win_bs[:, None], (32, NPT))

    Wfo = jnp.einsum('oik,kab->oabi', wout_mv, _BASIS,
                     precision=lax.Precision.HIGHEST)[0, 0]
    rx = jnp.zeros((8, 256), f32).at[0].set(Wfo.reshape(256))
    rs = jnp.zeros((8, 32), f32).at[0].set(wout_sm[0])
    mask2d = jnp.asarray(_MASK2Dn)

    full = lambda shape: pl.BlockSpec(shape, lambda b: (0,) * len(shape))
    out3 = pl.pallas_call(
        _body,
        grid=(B,),
        in_specs=[
            pl.BlockSpec((1, 8, NPT), lambda b: (b, 0, 0)),
            full((32, NPT)),
            full((256, 8)),
            full((256, 128)),
            full((L, 704, 256)),
            full((L, 704, 256)),
            full((L, 704, 32)),
            full((L, 288, 288)),
            full((L, 288, 288)),
            full((L, 576, 256)),
            full((L, 576, 256)),
            full((L, 576, 32)),
            full((L, 288, 256)),
            full((L, 288, 256)),
            full((L, 288, 32)),
            full((8, 256)),
            full((8, 32)),
        ],
        out_specs=pl.BlockSpec((1, 1, 128), lambda b: (b, 0, 0)),
        out_shape=jax.ShapeDtypeStruct((B, 1, 128), f32),
        compiler_params=pltpu.CompilerParams(
            dimension_semantics=("parallel",),
            vmem_limit_bytes=56 * 1024 * 1024,
        ),
    )(inpP, bs2d, A8, mask2d, wqkv_xh, wqkv_xl, wqkv_s, wao_h, wao_l,
      wm1_xh, wm1_xl, wm1_s, wm2_xh, wm2_xl, wm2_s, rx, rs)
    return out3[:, 0, :1]


# speedup vs baseline: 1.3370x; 1.3370x over previous
import numpy as np
import jax
import jax.numpy as jnp
from jax import lax
from jax.experimental import pallas as pl
from jax.experimental.pallas import tpu as pltpu

_BLADES = [(), (0,), (1,), (2,), (3,),
           (0, 1), (0, 2), (0, 3), (1, 2), (1, 3), (2, 3),
           (0, 1, 2), (0, 1, 3), (0, 2, 3), (1, 2, 3),
           (0, 1, 2, 3)]
_IDX = {b: i for i, b in enumerate(_BLADES)}


def _blade_mul(a, b):
    lst = list(a) + list(b); sign = 1; n = len(lst)
    for _ in range(n):
        for j in range(n - 1):
            if lst[j] > lst[j + 1]:
                lst[j], lst[j + 1] = lst[j + 1], lst[j]; sign = -sign
    out, i = [], 0
    while i < len(lst):
        if i + 1 < len(lst) and lst[i] == lst[i + 1]:
            if lst[i] == 0:
                return 0, ()
            i += 2
        else:
            out.append(lst[i]); i += 1
    return sign, tuple(out)


_GPn = np.zeros((16, 16, 16), dtype=np.float32)
for _i, _a in enumerate(_BLADES):
    for _j, _b in enumerate(_BLADES):
        _s, _c = _blade_mul(_a, _b)
        if _s:
            _GPn[_i, _j, _IDX[_c]] = _s
_gr = np.array([len(b) for b in _BLADES])
_P = np.stack([np.diag((_gr == g).astype(np.float32)) for g in range(5)])
_L0 = _GPn[1].T
_BASISn = np.concatenate([_P, np.stack([_L0 @ _P[g] for g in range(4)])], 0)
_BASIS = jnp.asarray(_BASISn)
_INNERn = np.array([0.0 if 0 in b else 1.0 for b in _BLADES], np.float32)
_NONE0 = [i for i, b in enumerate(_BLADES) if 0 not in b]

_GP_TERMS = [[] for _ in range(16)]
for _i in range(16):
    for _j in range(16):
        _nz = np.nonzero(_GPn[_i, _j])[0]
        if _nz.size:
            _GP_TERMS[int(_nz[0])].append((_i, _j, float(_GPn[_i, _j, _nz[0]])))

_C, _S, _H = 16, 32, 8
_HQ, _HV = 24, 40
_SCALE = 1.0 / np.sqrt(8 * (_C // _H) + _S // _H)
_PREC = lax.Precision.HIGHEST

_r = np.arange(256)
_PERM_AO = (_r % 16) * 16 + _r // 16
_IDX_OUT288 = np.concatenate([_PERM_AO, 256 + np.arange(32)]).astype(np.int32)

_r = np.arange(512)
_IDX_M1 = np.concatenate([(_r % 32) * 16 + _r // 32,
                          512 + np.arange(64)]).astype(np.int32)

_IDX_QKV = np.full((704,), 768 + 96, np.int32)
for _h in range(8):
    for _ci in range(2):
        for _t in range(8):
            _IDX_QKV[_h * 24 + _ci * 8 + _t] = (2 * _h + _ci) * 16 + _NONE0[_t]
            _IDX_QKV[192 + _h * 24 + _ci * 8 + _t] = (16 + 2 * _h + _ci) * 16 + _NONE0[_t]
        for _b in range(16):
            _IDX_QKV[384 + _h * 40 + _ci * 16 + _b] = (32 + 2 * _h + _ci) * 16 + _b
    for _d in range(4):
        _IDX_QKV[_h * 24 + 16 + _d] = 768 + _h * 4 + _d
        _IDX_QKV[192 + _h * 24 + 16 + _d] = 768 + 32 + _h * 4 + _d
        _IDX_QKV[384 + _h * 40 + 32 + _d] = 768 + 64 + _h * 4 + _d

_COL_AO = np.zeros((288,), np.int32)
for _h in range(8):
    for _ci in range(2):
        for _b in range(16):
            _COL_AO[_h * 36 + _ci * 16 + _b] = _b * 16 + (2 * _h + _ci)
    for _d in range(4):
        _COL_AO[_h * 36 + 32 + _d] = 256 + _h * 4 + _d

_QSCALEn = np.ones((1, 704, 1), np.float32)
_QSCALEn[:, :192, :] = _SCALE

_MASKROWn = np.repeat(_INNERn, 16).astype(np.float32)
_MASK2Dn = np.broadcast_to(_MASKROWn[:, None], (256, 128)).copy()


def _eq_canon(wmv, wsm, wms, wss):
    L, O, I, _ = wmv.shape
    So, Si = wss.shape[1], wss.shape[2]
    Wf = jnp.einsum('loik,kab->loabi', wmv, _BASIS,
                    precision=lax.Precision.HIGHEST)
    cx_mv = Wf.reshape(L, O * 16, 16 * I)
    cx_s = jnp.zeros((L, So, 16, I), wms.dtype).at[:, :, 0, :].set(wms)
    cx = jnp.concatenate([cx_mv, cx_s.reshape(L, So, 16 * I)], 1)
    cs_mv = jnp.zeros((L, O, 16, Si), wsm.dtype).at[:, :, 0, :].set(wsm)
    cs = jnp.concatenate([cs_mv.reshape(L, O * 16, Si), wss], 1)
    return cx, cs


def _take_rows(m, idx):
    z = jnp.concatenate([m, jnp.zeros_like(m[:, :1])], 1)
    return jnp.take(z, jnp.asarray(idx), axis=1)


def _take_cols(m, idx):
    z = jnp.concatenate([m, jnp.zeros_like(m[:, :, :1])], 2)
    return jnp.take(z, jnp.asarray(idx), axis=2)


def _body(inp_ref, bs_ref, a8_ref, mask_ref,
          wqkvxh_ref, wqkvxl_ref, wqkvs_ref, waoh_ref, waol_ref,
          wm1xh_ref, wm1xl_ref, wm1s_ref, wm2xh_ref, wm2xl_ref,
          wm2s_ref, rx_ref, rs_ref, out_ref):
    NPT = inp_ref.shape[2]
    L = wqkvs_ref.shape[0]
    f32 = jnp.float32
    bf16 = jnp.bfloat16

    def mm(a, b):
        return lax.dot_general(a, b, (((1,), (0,)), ((), ())),
                               precision=_PREC, preferred_element_type=f32)

    def mmT(a, b):
        return lax.dot_general(a, b, (((0,), (0,)), ((), ())),
                               precision=_PREC, preferred_element_type=f32)

    def split(x):
        hi = x.astype(bf16)
        lo = (x - hi.astype(f32)).astype(bf16)
        return hi, lo

    def bdot(a, b):
        return lax.dot_general(a, b, (((1,), (0,)), ((), ())),
                               preferred_element_type=f32)

    def bdotT(a, b):
        return lax.dot_general(a, b, (((0,), (0,)), ((), ())),
                               preferred_element_type=f32)

    def mm3(ah, al, b):
        bh, bl = split(b)
        return (bdot(ah, bl) + bdot(al, bh)) + bdot(ah, bh)

    def mm3T(a, b):
        ah, al = split(a)
        bh, bl = split(b)
        return (bdotT(ah, bl) + bdotT(al, bh)) + bdotT(ah, bh)

    mask1 = mask_ref[:, 0:1]

    def ln(X, Sc):
        f = jnp.sum(X * X * mask1, axis=0, keepdims=True) * (1.0 / 16.0)
        g = jnp.sum(Sc * Sc, axis=0, keepdims=True) * (1.0 / 32.0)
        return X * lax.rsqrt(f + 1e-6), Sc * lax.rsqrt(g + 1e-6)

    X = mm(a8_ref[...], inp_ref[0])
    Sc = bs_ref[...]

    def layer(l, carry):
        X, Sc = carry
        Xn, Sn = ln(X, Sc)
        qkv = mm3(wqkvxh_ref[l], wqkvxl_ref[l], Xn) \
            + mm(wqkvs_ref[l], Sn)
        qkvh, qkvl = split(qkv)
        one = jnp.ones((1, NPT), bf16)
        zero = jnp.zeros((1, NPT), bf16)
        outs = []
        for h in range(_H):
            q = lax.slice(qkv, (h * _HQ, 0), (h * _HQ + _HQ, NPT))
            k = lax.slice(qkv, (192 + h * _HQ, 0), (192 + h * _HQ + _HQ, NPT))
            qh = lax.slice(qkvh, (h * _HQ, 0), (h * _HQ + _HQ, NPT))
            ql = lax.slice(qkvl, (h * _HQ, 0), (h * _HQ + _HQ, NPT))
            kh = lax.slice(qkvh, (192 + h * _HQ, 0), (192 + h * _HQ + _HQ, NPT))
            kl = lax.slice(qkvl, (192 + h * _HQ, 0), (192 + h * _HQ + _HQ, NPT))
            vh = lax.slice(qkvh, (384 + h * _HV, 0), (384 + h * _HV + 36, NPT))
            vl = lax.slice(qkvl, (384 + h * _HV, 0), (384 + h * _HV + 36, NPT))
            qn2 = jnp.sum(q * q, axis=0, keepdims=True)
            kn2 = jnp.sum(k * k, axis=0, keepdims=True)
            bnd = jnp.sqrt(qn2 * jnp.max(kn2))
            kcat = jnp.concatenate([kh, kh, kl], axis=0)
            qcat = jnp.concatenate([qh, ql, qh], axis=0)
            sc = bdotT(kcat, qcat)
            p = jnp.exp(sc - bnd)
            ph, pl_ = split(p)
            vh1 = jnp.concatenate([vh, one], axis=0)
            vl1 = jnp.concatenate([vl, zero], axis=0)
            o = (bdot(vh1, pl_) + bdot(vl1, ph)) + bdot(vh1, ph)
            den = lax.slice(o, (36, 0), (37, NPT))
            o = lax.slice(o, (0, 0), (36, NPT)) * (1.0 / den)
            outs.append(o)
        O = jnp.concatenate(outs, axis=0)
        D = mm3(waoh_ref[l], waol_ref[l], O)
        X = X + lax.slice(D, (0, 0), (256, NPT))
        Sc = Sc + lax.slice(D, (256, 0), (288, NPT))

        Xn, Sn = ln(X, Sc)
        H1 = mm3(wm1xh_ref[l], wm1xl_ref[l], Xn) \
            + mm(wm1s_ref[l], Sn)
        gps = []
        for kk in range(16):
            acc = None
            for (i, j, s) in _GP_TERMS[kk]:
                t = (lax.slice(H1, (i * 32, 0), (i * 32 + 16, NPT))
                     * lax.slice(H1, (j * 32 + 16, 0), (j * 32 + 32, NPT)))
                t = t if s > 0 else -t
                acc = t if acc is None else acc + t
            gps.append(acc)
        gate = jax.nn.gelu(gps[0])
        gp = jnp.concatenate([g * gate for g in gps], axis=0)
        sh = (lax.slice(H1, (512, 0), (544, NPT))
              * jax.nn.gelu(lax.slice(H1, (544, 0), (576, NPT))))
        D = mm3(wm2xh_ref[l], wm2xl_ref[l], gp) \
            + mm(wm2s_ref[l], sh)
        X = X + lax.slice(D, (0, 0), (256, NPT))
        Sc = Sc + lax.slice(D, (256, 0), (288, NPT))
        return (X, Sc)

    X, Sc = lax.fori_loop(0, L, layer, (X, Sc))
    val = mm(rx_ref[...], X) + mm(rs_ref[...], Sc)
    mean = jnp.sum(lax.slice(val, (0, 0), (1, NPT)), axis=1,
                   keepdims=True) * (1.0 / NPT)
    out_ref[0] = jnp.broadcast_to(mean, (1, 128))


def kernel(inputs, win_mv, win_ms, win_bs,
           a_qkv_wmv, a_qkv_wsm, a_qkv_wms, a_qkv_wss,
           a_out_wmv, a_out_wsm, a_out_wms, a_out_wss,
           m1_wmv, m1_wsm, m1_wms, m1_wss,
           m2_wmv, m2_wsm, m2_wms, m2_wss,
           wout_mv, wout_sm):
    f32 = jnp.float32
    B, NPT, _ = inputs.shape
    L = a_qkv_wmv.shape[0]

    def wsplit(w):
        hi = w.astype(jnp.bfloat16)
        lo = (w - hi.astype(f32)).astype(jnp.bfloat16)
        return hi, lo

    cx, cs = _eq_canon(a_qkv_wmv, a_qkv_wsm, a_qkv_wms, a_qkv_wss)
    wqkv_xh, wqkv_xl = wsplit(_take_rows(cx, _IDX_QKV) * jnp.asarray(_QSCALEn))
    wqkv_s = _take_rows(cs, _IDX_QKV) * jnp.asarray(_QSCALEn)

    cx, cs = _eq_canon(a_out_wmv, a_out_wsm, a_out_wms, a_out_wss)
    w_full = jnp.concatenate([cx, cs], axis=2)
    wao_h, wao_l = wsplit(
        _take_cols(_take_rows(w_full, _IDX_OUT288), _COL_AO))

    cx, cs = _eq_canon(m1_wmv, m1_wsm, m1_wms, m1_wss)
    wm1_xh, wm1_xl = wsplit(_take_rows(cx, _IDX_M1))
    wm1_s = _take_rows(cs, _IDX_M1)

    cx, cs = _eq_canon(m2_wmv, m2_wsm, m2_wms, m2_wss)
    wm2_xh, wm2_xl = wsplit(_take_rows(cx, _IDX_OUT288))
    wm2_s = _take_rows(cs, _IDX_OUT288)

    W2d = jnp.einsum('ok,kab->aob', win_mv[:, 0, :], _BASIS,
                     precision=lax.Precision.HIGHEST).reshape(256, 16)
    A8 = jnp.stack([-W2d[:, 13], W2d[:, 12], -W2d[:, 11], W2d[:, 14]]
                   + [jnp.zeros((256,), f32)] * 4, axis=1)
    inpP = jnp.concatenate([jnp.swapaxes(inputs, 1, 2),
                            jnp.ones((B, 1, NPT), f32),
                            jnp.zeros((B, 4, NPT), f32)], axis=1)
    bs2d = jnp.broadcast_to(win_bs[:, None], (32, NPT))

    Wfo = jnp.einsum('oik,kab->oabi', wout_mv, _BASIS,
                     precision=lax.Precision.HIGHEST)[0, 0]
    rx = jnp.zeros((8, 256), f32).at[0].set(Wfo.reshape(256))
    rs = jnp.zeros((8, 32), f32).at[0].set(wout_sm[0])
    mask2d = jnp.asarray(_MASK2Dn)

    full = lambda shape: pl.BlockSpec(shape, lambda b: (0,) * len(shape))
    out3 = pl.pallas_call(
        _body,
        grid=(B,),
        in_specs=[
            pl.BlockSpec((1, 8, NPT), lambda b: (b, 0, 0)),
            full((32, NPT)),
            full((256, 8)),
            full((256, 128)),
            full((L, 704, 256)),
            full((L, 704, 256)),
            full((L, 704, 32)),
            full((L, 288, 288)),
            full((L, 288, 288)),
            full((L, 576, 256)),
            full((L, 576, 256)),
            full((L, 576, 32)),
            full((L, 288, 256)),
            full((L, 288, 256)),
            full((L, 288, 32)),
            full((8, 256)),
            full((8, 32)),
        ],
        out_specs=pl.BlockSpec((1, 1, 128), lambda b: (b, 0, 0)),
        out_shape=jax.ShapeDtypeStruct((B, 1, 128), f32),
        compiler_params=pltpu.CompilerParams(
            dimension_semantics=("parallel",),
            vmem_limit_bytes=56 * 1024 * 1024,
        ),
    )(inpP, bs2d, A8, mask2d, wqkv_xh, wqkv_xl, wqkv_s, wao_h, wao_l,
      wm1_xh, wm1_xl, wm1_s, wm2_xh, wm2_xl, wm2_s, rx, rs)
    return out3[:, 0, :1]
